# Initial kernel scaffold; baseline (speedup 1.0000x reference)
import jax
import jax.numpy as jnp
from jax.experimental import pallas as pl


def kernel(sequence_ids, item_embeddings, W_attn, b_attn, W_gate, b_gate, gamma, beta, W_proj, b_proj):
    raise NotImplementedError("write your pallas kernel here")


# speedup vs baseline: 1.7890x; 1.7890x over previous
import functools

import jax
import jax.numpy as jnp
from jax import lax
from jax.experimental import pallas as pl
from jax.experimental.pallas import tpu as pltpu
from jax.experimental.pallas import tpu_sc as plsc

V = 100000
B = 1024
L = 50
D = 64
K = 10
CHUNK = 128
VT = 1024
NT = 98
VPAD = NT * VT
NCHUNK = VPAD // CHUNK
NEG = -1.0e30
NEGINIT = -3.0e38

NC, NS = 2, 16
NW = NC * NS
RW = B // NW
LP = 64

_mesh = plsc.VectorSubcoreMesh(core_axis_name="c", subcore_axis_name="s")


def _iota16():
    return lax.iota(jnp.int32, 16)


@functools.partial(
    pl.kernel,
    mesh=_mesh,
    out_type=jax.ShapeDtypeStruct((B, D), jnp.float32),
    scratch_types=[
        pltpu.VMEM((RW, LP), jnp.int32),
        pltpu.VMEM((16, 128), jnp.int32),
        pltpu.VMEM((8, 16), jnp.int32),
        pltpu.VMEM((16, D), jnp.float32),
        pltpu.VMEM((RW * LP // 2, D), jnp.float32),
        pltpu.VMEM((RW, D), jnp.float32),
        pltpu.SemaphoreType.DMA,
    ],
    compiler_params=pltpu.CompilerParams(needs_layout_passes=False, use_tc_tiling_on_sc=False),
)
def _pool_sc(ids_hbm, table_hbm, out_hbm, ids_v, gidx_v, zidx_v, t0_v,
             rows_v, out_v, sem):
    wid = lax.axis_index("s") * NC + lax.axis_index("c")
    base = wid * RW
    pltpu.sync_copy(ids_hbm.at[pl.ds(base, RW)], ids_v)

    zidx_v[0, :] = jnp.zeros((16,), jnp.int32)
    t0c = pltpu.async_copy(table_hbm.at[zidx_v.at[0]], t0_v, sem)

    def build(r, _):
        for g in range(LP // 16):
            ids = ids_v[r, pl.ds(g * 16, 16)]
            shifted = jnp.maximum(ids - 1, 0)
            p = r * LP + g * 16
            gidx_v[p // 128, pl.ds(p % 128, 16)] = shifted
        return 0

    lax.fori_loop(0, RW, build, 0)
    t0c.wait()

    for h in range(2):
        copies = [
            pltpu.async_copy(
                table_hbm.at[gidx_v.at[h * 8 + q]],
                rows_v.at[pl.ds(q * 128, 128)], sem)
            for q in range(8)
        ]
        for c in copies:
            c.wait()

        def pool_row(rr, _):
            r = h * (RW // 2) + rr

            def accum(l, acc):
                row = rr * LP + l
                return tuple(
                    acc[j] + rows_v[row, pl.ds(j * 16, 16)]
                    for j in range(D // 16))

            acc = lax.fori_loop(
                0, LP, accum,
                tuple(jnp.zeros((16,), jnp.float32) for _ in range(D // 16)))
            cnt = jnp.float32(0.0)
            for g in range(LP // 16):
                ids = ids_v[r, pl.ds(g * 16, 16)]
                cnt = cnt + jnp.sum(jnp.where(ids != 0, jnp.float32(1.0),
                                              jnp.float32(0.0)))
            cntv = jnp.full((16,), cnt)
            inv = jnp.full((16,), jnp.float32(1.0)) / jnp.maximum(
                cntv, jnp.float32(1.0))
            ninv = jnp.float32(float(LP)) - cntv
            for j in range(D // 16):
                out_v[r, pl.ds(j * 16, 16)] = (
                    acc[j] - ninv * t0_v[0, pl.ds(j * 16, 16)]) * inv
            return 0

        lax.fori_loop(0, RW // 2, pool_row, 0)

    pltpu.sync_copy(out_v, out_hbm.at[pl.ds(base, RW)])


def _sims_body(urep_ref, table_ref, sims_ref, cmt_ref):
    j = pl.program_id(0)
    u = urep_ref[...]
    t = table_ref[...]
    s = lax.dot_general(u, t, (((1,), (1,)), ((), ())),
                        preferred_element_type=jnp.float32)

    def _mask(x):
        col = j * VT + lax.broadcasted_iota(jnp.int32, (B, VT), 1)
        return jnp.where(col < V, x, jnp.float32(NEG))

    s = lax.cond(j == NT - 1, _mask, lambda x: x, s)
    sims_ref[...] = s
    cm = jnp.concatenate(
        [jnp.max(s[:, g * CHUNK:(g + 1) * CHUNK], axis=1, keepdims=True)
         for g in range(VT // CHUNK)], axis=1)
    cmt_ref[...] = lax.transpose(cm, (1, 0))


def _sims_tc(urep, table):
    return pl.pallas_call(
        _sims_body,
        grid=(NT,),
        in_specs=[
            pl.BlockSpec((B, D), lambda j: (0, 0)),
            pl.BlockSpec((VT, D), lambda j: (j, 0)),
        ],
        out_specs=[
            pl.BlockSpec((B, VT), lambda j: (0, j)),
            pl.BlockSpec((VT // CHUNK, B), lambda j: (j, 0)),
        ],
        out_shape=[
            jax.ShapeDtypeStruct((B, VPAD), jnp.float32),
            jax.ShapeDtypeStruct((NCHUNK, B), jnp.float32),
        ],
        compiler_params=pltpu.CompilerParams(
            dimension_semantics=("arbitrary",)),
    )(urep, table)


def _topc_body(cmt_ref, topc_ref):
    cmv = cmt_ref[...]
    riota = lax.broadcasted_iota(jnp.int32, (NCHUNK, B), 0)
    topc_ref[pl.ds(K, 16 - K), :] = jnp.full((16 - K, B), NCHUNK - 1,
                                             jnp.int32)
    for k in range(K):
        m = jnp.max(cmv, axis=0, keepdims=True)
        hit = cmv == m
        idx = jnp.min(jnp.where(hit, riota, NCHUNK), axis=0,
                      keepdims=True)
        topc_ref[pl.ds(k, 1), :] = idx
        cmv = jnp.where(riota == idx, jnp.float32(NEGINIT), cmv)


def _topc_tc(cmt):
    return pl.pallas_call(
        _topc_body,
        out_shape=jax.ShapeDtypeStruct((16, B), jnp.int32),
    )(cmt)


@functools.partial(
    pl.kernel,
    mesh=_mesh,
    out_type=jax.ShapeDtypeStruct((B, D), jnp.float32),
    scratch_types=[
        pltpu.VMEM((RW, 16), jnp.int32),
        pltpu.VMEM((4, 128), jnp.int32),
        pltpu.VMEM((8, 16), jnp.int32),
        pltpu.VMEM((RW * 16, CHUNK), jnp.float32),
        pltpu.VMEM((RW * 16, D), jnp.float32),
        pltpu.VMEM((D,), jnp.float32),
        pltpu.VMEM((RW, D), jnp.float32),
        pltpu.SemaphoreType.DMA,
    ],
    compiler_params=pltpu.CompilerParams(needs_layout_passes=False, use_tc_tiling_on_sc=False),
)
def _retrieve_sc(sims2_hbm, topc_hbm, table_hbm, wa_hbm, out_hbm,
                 tc_v, gidx_v, eidx_v, cand_v, emb_v, wa_v, out_v, sem):
    wid = lax.axis_index("s") * NC + lax.axis_index("c")
    base = wid * RW
    pltpu.sync_copy(topc_hbm.at[pl.ds(base, RW)], tc_v)
    pltpu.sync_copy(wa_hbm, wa_v)

    def build(r, _):
        gidx_v[r // 8, pl.ds((r % 8) * 16, 16)] = (
            tc_v[r] + (base + r) * NCHUNK)
        return 0

    lax.fori_loop(0, RW, build, 0)

    copies = [
        pltpu.async_copy(sims2_hbm.at[gidx_v.at[q]],
                         cand_v.at[pl.ds(q * 128, 128)], sem)
        for q in range(4)
    ]
    for c in copies:
        c.wait()

    iota = _iota16()

    def select_row(r, _):
        tcrow = tc_v[r]

        def merge_chunk(c, carry):
            tv, ti = carry
            cid = jnp.sum(jnp.where(iota == c, tcrow, 0))
            colbase = cid * CHUNK
            for g in range(CHUNK // 16):
                val = cand_v[r * 16 + c, pl.ds(g * 16, 16)]
                col = colbase + g * 16 + iota
                sv, si = plsc.sort_key_val(val, col, descending=False)
                keep = tv >= sv
                mv = jnp.maximum(tv, sv)
                mi = jnp.where(keep, ti, si)
                tv, ti = plsc.sort_key_val(mv, mi, descending=True)
            return tv, ti

        tv, ti = lax.fori_loop(
            0, 16, merge_chunk,
            (jnp.full((16,), NEGINIT), jnp.zeros((16,), jnp.int32)))

        eidx_v[0, :] = ti
        pltpu.async_copy(table_hbm.at[eidx_v.at[0]],
                         emb_v.at[pl.ds(r * 16, 16)], sem).wait()

        sv = jnp.full((16,), NEGINIT)
        for k in range(K):
            acc = jnp.zeros((16,), jnp.float32)
            for j in range(D // 16):
                acc = acc + (emb_v[r * 16 + k, pl.ds(j * 16, 16)]
                             * wa_v[pl.ds(j * 16, 16)])
            sk = jnp.sum(acc)
            sv = jnp.where(iota == k, sk, sv)
        m = jnp.max(sv)
        e = jnp.exp(sv - m)
        z = jnp.sum(e)
        w = e / jnp.full((16,), z)

        for j in range(D // 16):
            acc = jnp.zeros((16,), jnp.float32)
            for k in range(K):
                wk = jnp.sum(jnp.where(iota == k, w, jnp.float32(0.0)))
                acc = acc + wk * emb_v[r * 16 + k, pl.ds(j * 16, 16)]
            out_v[r, pl.ds(j * 16, 16)] = acc
        return 0

    lax.fori_loop(0, RW, select_row, 0)
    pltpu.sync_copy(out_v, out_hbm.at[pl.ds(base, RW)])


def _proj_body(urep_ref, rrep_ref, wg_ref, bg_ref, gamma_ref, beta_ref,
               wp_ref, bp_ref, out_ref, fused_ref):
    j = pl.program_id(0)

    @pl.when(j == 0)
    def _():
        u = urep_ref[...]
        r = rrep_ref[...]
        wg = wg_ref[...]
        glin = (lax.dot_general(u, wg[0:D, :],
                                (((1,), (0,)), ((), ())),
                                preferred_element_type=jnp.float32)
                + lax.dot_general(r, wg[D:2 * D, :],
                                  (((1,), (0,)), ((), ())),
                                  preferred_element_type=jnp.float32)
                + bg_ref[0:1, :])
        gate = jax.nn.sigmoid(glin)
        fused = gate * u + (jnp.float32(1.0) - gate) * r
        mu = jnp.mean(fused, axis=-1, keepdims=True)
        dlt = fused - mu
        var = jnp.mean(dlt * dlt, axis=-1, keepdims=True)
        inv = lax.rsqrt(var + jnp.float32(1e-5))
        fused_ref[...] = (dlt * inv * gamma_ref[0:1, :] + beta_ref[0:1, :])

    out_ref[...] = (
        lax.dot_general(fused_ref[...], wp_ref[...], (((1,), (0,)), ((), ())),
                        preferred_element_type=jnp.float32)
        + bp_ref[0:1, :])


def _proj_tc(urep, rrep, wg, bg, gamma, beta, wp, bp):
    return pl.pallas_call(
        _proj_body,
        grid=(NT,),
        in_specs=[
            pl.BlockSpec((B, D), lambda j: (0, 0)),
            pl.BlockSpec((B, D), lambda j: (0, 0)),
            pl.BlockSpec((2 * D, D), lambda j: (0, 0)),
            pl.BlockSpec((8, D), lambda j: (0, 0)),
            pl.BlockSpec((8, D), lambda j: (0, 0)),
            pl.BlockSpec((8, D), lambda j: (0, 0)),
            pl.BlockSpec((D, VT), lambda j: (0, j)),
            pl.BlockSpec((8, VT), lambda j: (0, j)),
        ],
        out_specs=pl.BlockSpec((B, VT), lambda j: (0, j)),
        out_shape=jax.ShapeDtypeStruct((B, V), jnp.float32),
        scratch_shapes=[pltpu.VMEM((B, D), jnp.float32)],
        compiler_params=pltpu.CompilerParams(
            dimension_semantics=("arbitrary",)),
    )(urep, rrep, wg, bg, gamma, beta, wp, bp)


_DBG = "full"


def _jnp_pool(sequence_ids, table):
    seq_ids = jnp.clip(sequence_ids - 1, 0, None)
    pm = sequence_ids == 0
    se = jnp.take(table, seq_ids, axis=0)
    valid = (~pm).astype(jnp.float32)[..., None]
    counts = jnp.clip(jnp.sum(valid, axis=1), 1.0, None)
    return jnp.sum(se * valid, axis=1) / counts


def _jnp_simstopc(urep, table):
    sims = urep @ table.T
    sims_p = jnp.concatenate(
        [sims, jnp.full((B, VPAD - V), NEG, jnp.float32)], axis=1)
    cm = sims_p.reshape(B, NCHUNK, CHUNK).max(-1)
    _, tc = jax.lax.top_k(cm, K)
    tc16 = jnp.concatenate(
        [tc, jnp.full((B, 16 - K), NCHUNK - 1, jnp.int32)], axis=1)
    return sims_p, tc16


def _jnp_rrep(urep, table, W_attn):
    sims = urep @ table.T
    _, ridx = jax.lax.top_k(sims, K)
    re = jnp.take(table, ridx, axis=0)
    scores = (re @ W_attn)[..., 0]
    w = jax.nn.softmax(scores, axis=1)
    return jnp.sum(re * w[..., None], axis=1)


def _jnp_rrep_from(sims_p, tc16, table, W_attn):
    cols = tc16[:, :K, None] * CHUNK + jnp.arange(CHUNK)[None, None, :]
    cols = cols.reshape(B, K * CHUNK)
    cand = jnp.take_along_axis(sims_p, cols, axis=1)
    _, ci = jax.lax.top_k(cand, K)
    ridx = jnp.take_along_axis(cols, ci, axis=1)
    re = jnp.take(table, ridx, axis=0)
    scores = (re @ W_attn)[..., 0]
    w = jax.nn.softmax(scores, axis=1)
    return jnp.sum(re * w[..., None], axis=1)


def _jnp_logits(urep, rrep, W_gate, b_gate, gamma, beta, W_proj, b_proj):
    fi = jnp.concatenate([urep, rrep], axis=-1)
    gate = jax.nn.sigmoid(fi @ W_gate + b_gate)
    fused = gate * urep + (1.0 - gate) * rrep
    mu = jnp.mean(fused, axis=-1, keepdims=True)
    var = jnp.var(fused, axis=-1, keepdims=True)
    fused = (fused - mu) / jnp.sqrt(var + 1e-5) * gamma + beta
    return fused @ W_proj + b_proj


def kernel(sequence_ids, item_embeddings, W_attn, b_attn, W_gate, b_gate,
           gamma, beta, W_proj, b_proj):
    if _DBG != "full":
        ids_p = jnp.pad(sequence_ids.astype(jnp.int32), ((0, 0), (0, LP - L)))
        if _DBG == "A":
            urep = _pool_sc(ids_p, item_embeddings)
        else:
            urep = _jnp_pool(sequence_ids, item_embeddings)
        if _DBG == "B":
            sims, cmt = _sims_tc(urep, item_embeddings)
            topc = _topc_tc(cmt)
            rrep = _jnp_rrep_from(sims, topc.T, item_embeddings, W_attn)
        elif _DBG == "B3":
            sims, cmt = _sims_tc(urep, item_embeddings)
            _, tc = jax.lax.top_k(cmt.T, K)
            tc16 = jnp.concatenate(
                [tc, jnp.full((B, 16 - K), NCHUNK - 1, jnp.int32)], axis=1)
            rrep = _jnp_rrep_from(sims, tc16, item_embeddings, W_attn)
        elif _DBG == "B2":
            sims, _cmt = _sims_tc(urep, item_embeddings)
            cm = sims.reshape(B, NCHUNK, CHUNK).max(-1)
            _, tc = jax.lax.top_k(cm, K)
            tc16 = jnp.concatenate(
                [tc, jnp.full((B, 16 - K), NCHUNK - 1, jnp.int32)], axis=1)
            rrep = _jnp_rrep_from(sims, tc16, item_embeddings, W_attn)
        elif _DBG == "C":
            sims_p, tc16 = _jnp_simstopc(urep, item_embeddings)
            rrep = _retrieve_sc(sims_p.reshape(B * NCHUNK, CHUNK), tc16,
                                item_embeddings, W_attn.reshape(D))
        else:
            rrep = _jnp_rrep(urep, item_embeddings, W_attn)
        if _DBG == "D":
            return _proj_tc(
                urep, rrep, W_gate,
                jnp.broadcast_to(b_gate.reshape(1, D), (8, D)),
                jnp.broadcast_to(gamma.reshape(1, D), (8, D)),
                jnp.broadcast_to(beta.reshape(1, D), (8, D)),
                W_proj,
                jnp.broadcast_to(b_proj.reshape(1, V), (8, V)))
        return _jnp_logits(urep, rrep, W_gate, b_gate, gamma, beta,
                           W_proj, b_proj)
    return _kernel_impl(sequence_ids, item_embeddings, W_attn, b_attn,
                        W_gate, b_gate, gamma, beta, W_proj, b_proj)


def _kernel_impl(sequence_ids, item_embeddings, W_attn, b_attn, W_gate,
                 b_gate, gamma, beta, W_proj, b_proj):
    ids_p = jnp.pad(sequence_ids.astype(jnp.int32), ((0, 0), (0, LP - L)))
    user_rep = _pool_sc(ids_p, item_embeddings)
    sims, cmt = _sims_tc(user_rep, item_embeddings)
    topc = _topc_tc(cmt)
    sims2 = sims.reshape(B * NCHUNK, CHUNK)
    topc_t = topc.T
    rrep = _retrieve_sc(sims2, topc_t, item_embeddings, W_attn.reshape(D))
    logits = _proj_tc(
        user_rep, rrep, W_gate,
        jnp.broadcast_to(b_gate.reshape(1, D), (8, D)),
        jnp.broadcast_to(gamma.reshape(1, D), (8, D)),
        jnp.broadcast_to(beta.reshape(1, D), (8, D)),
        W_proj,
        jnp.broadcast_to(b_proj.reshape(1, V), (8, V)))
    return logits


# speedup vs baseline: 2.0626x; 1.1529x over previous
import functools

import jax
import jax.numpy as jnp
from jax import lax
from jax.experimental import pallas as pl
from jax.experimental.pallas import tpu as pltpu
from jax.experimental.pallas import tpu_sc as plsc

V = 100000
B = 1024
L = 50
D = 64
K = 10
CHUNK = 128
VT = 1024
NT = 98
VPAD = NT * VT
NCHUNK = VPAD // CHUNK
NEG = -1.0e30
NEGINIT = -3.0e38

NC, NS = 2, 16
NW = NC * NS
RW = B // NW
LP = 64

_mesh = plsc.VectorSubcoreMesh(core_axis_name="c", subcore_axis_name="s")


def _iota16():
    return lax.iota(jnp.int32, 16)


@functools.partial(
    pl.kernel,
    mesh=_mesh,
    out_type=jax.ShapeDtypeStruct((B, D), jnp.float32),
    scratch_types=[
        pltpu.VMEM((RW, LP), jnp.int32),
        pltpu.VMEM((16, 128), jnp.int32),
        pltpu.VMEM((8, 16), jnp.int32),
        pltpu.VMEM((16, D), jnp.float32),
        pltpu.VMEM((RW * LP // 2, D), jnp.float32),
        pltpu.VMEM((RW, D), jnp.float32),
        pltpu.SemaphoreType.DMA,
    ],
    compiler_params=pltpu.CompilerParams(needs_layout_passes=False, use_tc_tiling_on_sc=False),
)
def _pool_sc(ids_hbm, table_hbm, out_hbm, ids_v, gidx_v, zidx_v, t0_v,
             rows_v, out_v, sem):
    wid = lax.axis_index("s") * NC + lax.axis_index("c")
    base = wid * RW
    pltpu.sync_copy(ids_hbm.at[pl.ds(base, RW)], ids_v)

    zidx_v[0, :] = jnp.zeros((16,), jnp.int32)
    t0c = pltpu.async_copy(table_hbm.at[zidx_v.at[0]], t0_v, sem)

    def build(r, _):
        for g in range(LP // 16):
            ids = ids_v[r, pl.ds(g * 16, 16)]
            shifted = jnp.maximum(ids - 1, 0)
            p = r * LP + g * 16
            gidx_v[p // 128, pl.ds(p % 128, 16)] = shifted
        return 0

    lax.fori_loop(0, RW, build, 0)
    t0c.wait()

    for h in range(2):
        copies = [
            pltpu.async_copy(
                table_hbm.at[gidx_v.at[h * 8 + q]],
                rows_v.at[pl.ds(q * 128, 128)], sem)
            for q in range(8)
        ]
        for c in copies:
            c.wait()

        def pool_row(rr, _):
            r = h * (RW // 2) + rr

            def accum(l, acc):
                row = rr * LP + l
                return tuple(
                    acc[j] + rows_v[row, pl.ds(j * 16, 16)]
                    for j in range(D // 16))

            acc = lax.fori_loop(
                0, LP, accum,
                tuple(jnp.zeros((16,), jnp.float32) for _ in range(D // 16)))
            cnt = jnp.float32(0.0)
            for g in range(LP // 16):
                ids = ids_v[r, pl.ds(g * 16, 16)]
                cnt = cnt + jnp.sum(jnp.where(ids != 0, jnp.float32(1.0),
                                              jnp.float32(0.0)))
            cntv = jnp.full((16,), cnt)
            inv = jnp.full((16,), jnp.float32(1.0)) / jnp.maximum(
                cntv, jnp.float32(1.0))
            ninv = jnp.float32(float(LP)) - cntv
            for j in range(D // 16):
                out_v[r, pl.ds(j * 16, 16)] = (
                    acc[j] - ninv * t0_v[0, pl.ds(j * 16, 16)]) * inv
            return 0

        lax.fori_loop(0, RW // 2, pool_row, 0)

    pltpu.sync_copy(out_v, out_hbm.at[pl.ds(base, RW)])


def _sims_body(urep_ref, table_ref, sims_ref, cmt_ref):
    j = pl.program_id(0)
    u = urep_ref[...]
    t = table_ref[...]
    s = lax.dot_general(u, t, (((1,), (1,)), ((), ())),
                        preferred_element_type=jnp.float32)

    def _mask(x):
        col = j * VT + lax.broadcasted_iota(jnp.int32, (B, VT), 1)
        return jnp.where(col < V, x, jnp.float32(NEG))

    s = lax.cond(j == NT - 1, _mask, lambda x: x, s)
    sims_ref[...] = s.reshape(B, VT // CHUNK, CHUNK)
    cm = jnp.concatenate(
        [jnp.max(s[:, g * CHUNK:(g + 1) * CHUNK], axis=1, keepdims=True)
         for g in range(VT // CHUNK)], axis=1)
    cmt_ref[...] = lax.transpose(cm, (1, 0))


def _sims_tc(urep, table):
    return pl.pallas_call(
        _sims_body,
        grid=(NT,),
        in_specs=[
            pl.BlockSpec((B, D), lambda j: (0, 0)),
            pl.BlockSpec((VT, D), lambda j: (j, 0)),
        ],
        out_specs=[
            pl.BlockSpec((B, VT // CHUNK, CHUNK), lambda j: (0, j, 0)),
            pl.BlockSpec((VT // CHUNK, B), lambda j: (j, 0)),
        ],
        out_shape=[
            jax.ShapeDtypeStruct((B, NCHUNK, CHUNK), jnp.float32),
            jax.ShapeDtypeStruct((NCHUNK, B), jnp.float32),
        ],
        compiler_params=pltpu.CompilerParams(
            dimension_semantics=("arbitrary",)),
    )(urep, table)


def _topc_body(cmt_ref, topc_ref):
    cmv = cmt_ref[...]
    riota = lax.broadcasted_iota(jnp.int32, (NCHUNK, B), 0)
    topc_ref[pl.ds(K, 16 - K), :] = jnp.full((16 - K, B), NCHUNK - 1,
                                             jnp.int32)
    for k in range(K):
        m = jnp.max(cmv, axis=0, keepdims=True)
        hit = cmv == m
        idx = jnp.min(jnp.where(hit, riota, NCHUNK), axis=0,
                      keepdims=True)
        topc_ref[pl.ds(k, 1), :] = idx
        cmv = jnp.where(riota == idx, jnp.float32(NEGINIT), cmv)


def _topc_tc(cmt):
    return pl.pallas_call(
        _topc_body,
        out_shape=jax.ShapeDtypeStruct((16, B), jnp.int32),
    )(cmt)


@functools.partial(
    pl.kernel,
    mesh=_mesh,
    out_type=jax.ShapeDtypeStruct((B, D), jnp.float32),
    scratch_types=[
        pltpu.VMEM((RW, 16), jnp.int32),
        pltpu.VMEM((4, 128), jnp.int32),
        pltpu.VMEM((8, 16), jnp.int32),
        pltpu.VMEM((RW * 16, CHUNK), jnp.float32),
        pltpu.VMEM((RW * 16, D), jnp.float32),
        pltpu.VMEM((D,), jnp.float32),
        pltpu.VMEM((RW, D), jnp.float32),
        pltpu.SemaphoreType.DMA,
    ],
    compiler_params=pltpu.CompilerParams(needs_layout_passes=False, use_tc_tiling_on_sc=False),
)
def _retrieve_sc(sims2_hbm, topc_hbm, table_hbm, wa_hbm, out_hbm,
                 tc_v, gidx_v, eidx_v, cand_v, emb_v, wa_v, out_v, sem):
    wid = lax.axis_index("s") * NC + lax.axis_index("c")
    base = wid * RW
    pltpu.sync_copy(topc_hbm.at[pl.ds(base, RW)], tc_v)
    pltpu.sync_copy(wa_hbm, wa_v)

    def build(r, _):
        gidx_v[r // 8, pl.ds((r % 8) * 16, 16)] = (
            tc_v[r] + (base + r) * NCHUNK)
        return 0

    lax.fori_loop(0, RW, build, 0)

    copies = [
        pltpu.async_copy(sims2_hbm.at[gidx_v.at[q]],
                         cand_v.at[pl.ds(q * 128, 128)], sem)
        for q in range(4)
    ]
    for c in copies:
        c.wait()

    iota = _iota16()

    def select_row(r, _):
        tcrow = tc_v[r]

        def merge_chunk(c, carry):
            tv, ti = carry
            cid = jnp.sum(jnp.where(iota == c, tcrow, 0))
            colbase = cid * CHUNK
            for g in range(CHUNK // 16):
                val = cand_v[r * 16 + c, pl.ds(g * 16, 16)]
                col = colbase + g * 16 + iota
                sv, si = plsc.sort_key_val(val, col, descending=False)
                keep = tv >= sv
                mv = jnp.maximum(tv, sv)
                mi = jnp.where(keep, ti, si)
                tv, ti = plsc.sort_key_val(mv, mi, descending=True)
            return tv, ti

        tv, ti = lax.fori_loop(
            0, 16, merge_chunk,
            (jnp.full((16,), NEGINIT), jnp.zeros((16,), jnp.int32)))

        eidx_v[0, :] = ti
        pltpu.async_copy(table_hbm.at[eidx_v.at[0]],
                         emb_v.at[pl.ds(r * 16, 16)], sem).wait()

        sv = jnp.full((16,), NEGINIT)
        for k in range(K):
            acc = jnp.zeros((16,), jnp.float32)
            for j in range(D // 16):
                acc = acc + (emb_v[r * 16 + k, pl.ds(j * 16, 16)]
                             * wa_v[pl.ds(j * 16, 16)])
            sk = jnp.sum(acc)
            sv = jnp.where(iota == k, sk, sv)
        m = jnp.max(sv)
        e = jnp.exp(sv - m)
        z = jnp.sum(e)
        w = e / jnp.full((16,), z)

        for j in range(D // 16):
            acc = jnp.zeros((16,), jnp.float32)
            for k in range(K):
                wk = jnp.sum(jnp.where(iota == k, w, jnp.float32(0.0)))
                acc = acc + wk * emb_v[r * 16 + k, pl.ds(j * 16, 16)]
            out_v[r, pl.ds(j * 16, 16)] = acc
        return 0

    lax.fori_loop(0, RW, select_row, 0)
    pltpu.sync_copy(out_v, out_hbm.at[pl.ds(base, RW)])


def _proj_body(urep_ref, rrep_ref, wg_ref, bg_ref, gamma_ref, beta_ref,
               wp_ref, bp_ref, out_ref, fused_ref):
    j = pl.program_id(0)

    @pl.when(j == 0)
    def _():
        u = urep_ref[...]
        r = rrep_ref[...]
        wg = wg_ref[...]
        glin = (lax.dot_general(u, wg[0:D, :],
                                (((1,), (0,)), ((), ())),
                                preferred_element_type=jnp.float32)
                + lax.dot_general(r, wg[D:2 * D, :],
                                  (((1,), (0,)), ((), ())),
                                  preferred_element_type=jnp.float32)
                + bg_ref[0:1, :])
        gate = jax.nn.sigmoid(glin)
        fused = gate * u + (jnp.float32(1.0) - gate) * r
        mu = jnp.mean(fused, axis=-1, keepdims=True)
        dlt = fused - mu
        var = jnp.mean(dlt * dlt, axis=-1, keepdims=True)
        inv = lax.rsqrt(var + jnp.float32(1e-5))
        fused_ref[...] = (dlt * inv * gamma_ref[0:1, :] + beta_ref[0:1, :])

    out_ref[...] = (
        lax.dot_general(fused_ref[...], wp_ref[...], (((1,), (0,)), ((), ())),
                        preferred_element_type=jnp.float32)
        + bp_ref[0:1, :])


def _proj_tc(urep, rrep, wg, bg, gamma, beta, wp, bp):
    return pl.pallas_call(
        _proj_body,
        grid=(NT,),
        in_specs=[
            pl.BlockSpec((B, D), lambda j: (0, 0)),
            pl.BlockSpec((B, D), lambda j: (0, 0)),
            pl.BlockSpec((2 * D, D), lambda j: (0, 0)),
            pl.BlockSpec((8, D), lambda j: (0, 0)),
            pl.BlockSpec((8, D), lambda j: (0, 0)),
            pl.BlockSpec((8, D), lambda j: (0, 0)),
            pl.BlockSpec((D, VT), lambda j: (0, j)),
            pl.BlockSpec((8, VT), lambda j: (0, j)),
        ],
        out_specs=pl.BlockSpec((B, VT), lambda j: (0, j)),
        out_shape=jax.ShapeDtypeStruct((B, V), jnp.float32),
        scratch_shapes=[pltpu.VMEM((B, D), jnp.float32)],
        compiler_params=pltpu.CompilerParams(
            dimension_semantics=("arbitrary",)),
    )(urep, rrep, wg, bg, gamma, beta, wp, bp)


def kernel(sequence_ids, item_embeddings, W_attn, b_attn, W_gate, b_gate,
           gamma, beta, W_proj, b_proj):
    ids_p = jnp.pad(sequence_ids.astype(jnp.int32), ((0, 0), (0, LP - L)))
    user_rep = _pool_sc(ids_p, item_embeddings)
    sims, cmt = _sims_tc(user_rep, item_embeddings)
    topc = _topc_tc(cmt)
    sims2 = sims.reshape(B * NCHUNK, CHUNK)
    topc_t = topc.T
    rrep = _retrieve_sc(sims2, topc_t, item_embeddings, W_attn.reshape(D))
    logits = _proj_tc(
        user_rep, rrep, W_gate,
        jnp.broadcast_to(b_gate.reshape(1, D), (8, D)),
        jnp.broadcast_to(gamma.reshape(1, D), (8, D)),
        jnp.broadcast_to(beta.reshape(1, D), (8, D)),
        W_proj,
        jnp.broadcast_to(b_proj.reshape(1, V), (8, V)))
    return logits


# speedup vs baseline: 2.0901x; 1.0134x over previous
import functools

import jax
import jax.numpy as jnp
from jax import lax
from jax.experimental import pallas as pl
from jax.experimental.pallas import tpu as pltpu
from jax.experimental.pallas import tpu_sc as plsc

V = 100000
B = 1024
L = 50
D = 64
K = 10
CHUNK = 128
VT = 1024
NT = 98
VPAD = NT * VT
NCHUNK = VPAD // CHUNK
NEG = -1.0e30
NEGINIT = -3.0e38

NC, NS = 2, 16
NW = NC * NS
RW = B // NW
LP = 64

_mesh = plsc.VectorSubcoreMesh(core_axis_name="c", subcore_axis_name="s")


def _iota16():
    return lax.iota(jnp.int32, 16)


@functools.partial(
    pl.kernel,
    mesh=_mesh,
    out_type=jax.ShapeDtypeStruct((B, D), jnp.float32),
    scratch_types=[
        pltpu.VMEM((RW, LP), jnp.int32),
        pltpu.VMEM((16, 128), jnp.int32),
        pltpu.VMEM((8, 16), jnp.int32),
        pltpu.VMEM((16, D), jnp.float32),
        pltpu.VMEM((RW * LP // 2, D), jnp.float32),
        pltpu.VMEM((RW, D), jnp.float32),
        pltpu.SemaphoreType.DMA,
    ],
    compiler_params=pltpu.CompilerParams(needs_layout_passes=False, use_tc_tiling_on_sc=False),
)
def _pool_sc(ids_hbm, table_hbm, out_hbm, ids_v, gidx_v, zidx_v, t0_v,
             rows_v, out_v, sem):
    wid = lax.axis_index("s") * NC + lax.axis_index("c")
    base = wid * RW
    pltpu.sync_copy(ids_hbm.at[pl.ds(base, RW)], ids_v)

    zidx_v[0, :] = jnp.zeros((16,), jnp.int32)
    t0c = pltpu.async_copy(table_hbm.at[zidx_v.at[0]], t0_v, sem)

    def build(r, _):
        for g in range(LP // 16):
            ids = ids_v[r, pl.ds(g * 16, 16)]
            shifted = jnp.maximum(ids - 1, 0)
            p = r * LP + g * 16
            gidx_v[p // 128, pl.ds(p % 128, 16)] = shifted
        return 0

    lax.fori_loop(0, RW, build, 0)
    t0c.wait()

    for h in range(2):
        copies = [
            pltpu.async_copy(
                table_hbm.at[gidx_v.at[h * 8 + q]],
                rows_v.at[pl.ds(q * 128, 128)], sem)
            for q in range(8)
        ]
        for c in copies:
            c.wait()

        def pool_row(rr, _):
            r = h * (RW // 2) + rr

            def accum(l, acc):
                row = rr * LP + l
                return tuple(
                    acc[j] + rows_v[row, pl.ds(j * 16, 16)]
                    for j in range(D // 16))

            acc = lax.fori_loop(
                0, LP, accum,
                tuple(jnp.zeros((16,), jnp.float32) for _ in range(D // 16)))
            cnt = jnp.float32(0.0)
            for g in range(LP // 16):
                ids = ids_v[r, pl.ds(g * 16, 16)]
                cnt = cnt + jnp.sum(jnp.where(ids != 0, jnp.float32(1.0),
                                              jnp.float32(0.0)))
            cntv = jnp.full((16,), cnt)
            inv = jnp.full((16,), jnp.float32(1.0)) / jnp.maximum(
                cntv, jnp.float32(1.0))
            ninv = jnp.float32(float(LP)) - cntv
            for j in range(D // 16):
                out_v[r, pl.ds(j * 16, 16)] = (
                    acc[j] - ninv * t0_v[0, pl.ds(j * 16, 16)]) * inv
            return 0

        lax.fori_loop(0, RW // 2, pool_row, 0)

    pltpu.sync_copy(out_v, out_hbm.at[pl.ds(base, RW)])


def _sims_body(urep_ref, table_ref, sims_ref, cmt_ref):
    j = pl.program_id(0)
    u = urep_ref[...]
    t = table_ref[...]
    s = lax.dot_general(u, t, (((1,), (1,)), ((), ())),
                        preferred_element_type=jnp.float32)

    def _mask(x):
        col = j * VT + lax.broadcasted_iota(jnp.int32, (B, VT), 1)
        return jnp.where(col < V, x, jnp.float32(NEG))

    s = lax.cond(j == NT - 1, _mask, lambda x: x, s)
    sims_ref[...] = s.reshape(B, VT // CHUNK, CHUNK)
    cm = jnp.concatenate(
        [jnp.max(s[:, g * CHUNK:(g + 1) * CHUNK], axis=1, keepdims=True)
         for g in range(VT // CHUNK)], axis=1)
    cmt_ref[...] = lax.transpose(cm, (1, 0))


def _sims_tc(urep, table):
    return pl.pallas_call(
        _sims_body,
        grid=(NT,),
        in_specs=[
            pl.BlockSpec((B, D), lambda j: (0, 0)),
            pl.BlockSpec((VT, D), lambda j: (j, 0)),
        ],
        out_specs=[
            pl.BlockSpec((B, VT // CHUNK, CHUNK), lambda j: (0, j, 0)),
            pl.BlockSpec((VT // CHUNK, B), lambda j: (j, 0)),
        ],
        out_shape=[
            jax.ShapeDtypeStruct((B, NCHUNK, CHUNK), jnp.float32),
            jax.ShapeDtypeStruct((NCHUNK, B), jnp.float32),
        ],
        compiler_params=pltpu.CompilerParams(
            dimension_semantics=("arbitrary",)),
    )(urep, table)


def _topc_body(cmt_ref, topc_ref):
    cmv = cmt_ref[...]
    riota = lax.broadcasted_iota(jnp.int32, (NCHUNK, B), 0)
    topc_ref[pl.ds(K, 16 - K), :] = jnp.full((16 - K, B), NCHUNK - 1,
                                             jnp.int32)
    for k in range(K):
        m = jnp.max(cmv, axis=0, keepdims=True)
        hit = cmv == m
        idx = jnp.min(jnp.where(hit, riota, NCHUNK), axis=0,
                      keepdims=True)
        topc_ref[pl.ds(k, 1), :] = idx
        cmv = jnp.where(riota == idx, jnp.float32(NEGINIT), cmv)


def _topc_tc(cmt):
    return pl.pallas_call(
        _topc_body,
        out_shape=jax.ShapeDtypeStruct((16, B), jnp.int32),
    )(cmt)


@functools.partial(
    pl.kernel,
    mesh=_mesh,
    out_type=jax.ShapeDtypeStruct((B, D), jnp.float32),
    scratch_types=[
        pltpu.VMEM((RW, 16), jnp.int32),
        pltpu.VMEM((4, 128), jnp.int32),
        pltpu.VMEM((8, 16), jnp.int32),
        pltpu.VMEM((RW * 16, CHUNK), jnp.float32),
        pltpu.VMEM((RW * 16, D), jnp.float32),
        pltpu.VMEM((D,), jnp.float32),
        pltpu.VMEM((RW, D), jnp.float32),
        pltpu.SemaphoreType.DMA,
    ],
    compiler_params=pltpu.CompilerParams(needs_layout_passes=False, use_tc_tiling_on_sc=False),
)
def _retrieve_sc(sims2_hbm, topc_hbm, table_hbm, wa_hbm, out_hbm,
                 tc_v, gidx_v, eidx_v, cand_v, emb_v, wa_v, out_v, sem):
    wid = lax.axis_index("s") * NC + lax.axis_index("c")
    base = wid * RW
    pltpu.sync_copy(topc_hbm.at[pl.ds(base, RW)], tc_v)
    pltpu.sync_copy(wa_hbm, wa_v)

    def build(r, _):
        gidx_v[r // 8, pl.ds((r % 8) * 16, 16)] = (
            tc_v[r] + (base + r) * NCHUNK)
        return 0

    lax.fori_loop(0, RW, build, 0)

    copies = [
        pltpu.async_copy(sims2_hbm.at[gidx_v.at[q]],
                         cand_v.at[pl.ds(q * 128, 128)], sem)
        for q in range(4)
    ]
    for c in copies:
        c.wait()

    iota = _iota16()

    def select_row(r, _):
        tcrow = tc_v[r]

        def merge_chunk(c, carry):
            tv, ti = carry
            cid = jnp.sum(jnp.where(iota == c, tcrow, 0))
            colbase = cid * CHUNK
            for g in range(CHUNK // 16):
                val = cand_v[r * 16 + c, pl.ds(g * 16, 16)]
                col = colbase + g * 16 + iota
                sv, si = plsc.sort_key_val(val, col, descending=False)
                keep = tv >= sv
                mv = jnp.maximum(tv, sv)
                mi = jnp.where(keep, ti, si)
                tv, ti = plsc.sort_key_val(mv, mi, descending=True)
            return tv, ti

        tv, ti = lax.fori_loop(
            0, K, merge_chunk,
            (jnp.full((16,), NEGINIT), jnp.zeros((16,), jnp.int32)))

        eidx_v[0, :] = ti
        pltpu.async_copy(table_hbm.at[eidx_v.at[0]],
                         emb_v.at[pl.ds(r * 16, 16)], sem).wait()

        sv = jnp.full((16,), NEGINIT)
        for k in range(K):
            acc = jnp.zeros((16,), jnp.float32)
            for j in range(D // 16):
                acc = acc + (emb_v[r * 16 + k, pl.ds(j * 16, 16)]
                             * wa_v[pl.ds(j * 16, 16)])
            sk = jnp.sum(acc)
            sv = jnp.where(iota == k, sk, sv)
        m = jnp.max(sv)
        e = jnp.exp(sv - m)
        z = jnp.sum(e)
        w = e / jnp.full((16,), z)

        for j in range(D // 16):
            acc = jnp.zeros((16,), jnp.float32)
            for k in range(K):
                wk = jnp.sum(jnp.where(iota == k, w, jnp.float32(0.0)))
                acc = acc + wk * emb_v[r * 16 + k, pl.ds(j * 16, 16)]
            out_v[r, pl.ds(j * 16, 16)] = acc
        return 0

    lax.fori_loop(0, RW, select_row, 0)
    pltpu.sync_copy(out_v, out_hbm.at[pl.ds(base, RW)])


def _proj_body(urep_ref, rrep_ref, wg_ref, bg_ref, gamma_ref, beta_ref,
               wp_ref, bp_ref, out_ref, fused_ref):
    j = pl.program_id(0)

    @pl.when(j == 0)
    def _():
        u = urep_ref[...]
        r = rrep_ref[...]
        wg = wg_ref[...]
        glin = (lax.dot_general(u, wg[0:D, :],
                                (((1,), (0,)), ((), ())),
                                preferred_element_type=jnp.float32)
                + lax.dot_general(r, wg[D:2 * D, :],
                                  (((1,), (0,)), ((), ())),
                                  preferred_element_type=jnp.float32)
                + bg_ref[0:1, :])
        gate = jax.nn.sigmoid(glin)
        fused = gate * u + (jnp.float32(1.0) - gate) * r
        mu = jnp.mean(fused, axis=-1, keepdims=True)
        dlt = fused - mu
        var = jnp.mean(dlt * dlt, axis=-1, keepdims=True)
        inv = lax.rsqrt(var + jnp.float32(1e-5))
        fused_ref[...] = (dlt * inv * gamma_ref[0:1, :] + beta_ref[0:1, :])

    out_ref[...] = (
        lax.dot_general(fused_ref[...], wp_ref[...], (((1,), (0,)), ((), ())),
                        preferred_element_type=jnp.float32)
        + bp_ref[0:1, :])


def _proj_tc(urep, rrep, wg, bg, gamma, beta, wp, bp):
    return pl.pallas_call(
        _proj_body,
        grid=(NT,),
        in_specs=[
            pl.BlockSpec((B, D), lambda j: (0, 0)),
            pl.BlockSpec((B, D), lambda j: (0, 0)),
            pl.BlockSpec((2 * D, D), lambda j: (0, 0)),
            pl.BlockSpec((8, D), lambda j: (0, 0)),
            pl.BlockSpec((8, D), lambda j: (0, 0)),
            pl.BlockSpec((8, D), lambda j: (0, 0)),
            pl.BlockSpec((D, VT), lambda j: (0, j)),
            pl.BlockSpec((8, VT), lambda j: (0, j)),
        ],
        out_specs=pl.BlockSpec((B, VT), lambda j: (0, j)),
        out_shape=jax.ShapeDtypeStruct((B, V), jnp.float32),
        scratch_shapes=[pltpu.VMEM((B, D), jnp.float32)],
        compiler_params=pltpu.CompilerParams(
            dimension_semantics=("arbitrary",)),
    )(urep, rrep, wg, bg, gamma, beta, wp, bp)


def kernel(sequence_ids, item_embeddings, W_attn, b_attn, W_gate, b_gate,
           gamma, beta, W_proj, b_proj):
    ids_p = jnp.pad(sequence_ids.astype(jnp.int32), ((0, 0), (0, LP - L)))
    user_rep = _pool_sc(ids_p, item_embeddings)
    sims, cmt = _sims_tc(user_rep, item_embeddings)
    topc = _topc_tc(cmt)
    sims2 = sims.reshape(B * NCHUNK, CHUNK)
    topc_t = topc.T
    rrep = _retrieve_sc(sims2, topc_t, item_embeddings, W_attn.reshape(D))
    logits = _proj_tc(
        user_rep, rrep, W_gate,
        jnp.broadcast_to(b_gate.reshape(1, D), (8, D)),
        jnp.broadcast_to(gamma.reshape(1, D), (8, D)),
        jnp.broadcast_to(beta.reshape(1, D), (8, D)),
        W_proj,
        jnp.broadcast_to(b_proj.reshape(1, V), (8, V)))
    return logits
